# Initial kernel scaffold; baseline (speedup 1.0000x reference)
import jax
import jax.numpy as jnp
from jax.experimental import pallas as pl


def kernel(x, lut):
    raise NotImplementedError("write your pallas kernel here")


# speedup vs baseline: 2.0583x; 2.0583x over previous
import jax
import jax.numpy as jnp
from jax import lax
from jax.experimental import pallas as pl
from jax.experimental.pallas import tpu as pltpu
from jax.experimental.pallas import tpu_sc as plsc

LUT_SIZE = 1000000
KEY_DIM = 32
DIGESTS = 2
HASH_C = 73244475

NC = 2
NS = 16
NW = NC * NS
LANES = 16

CH = 1024
GB = 128
K = CH // GB


def _wrap64_py(v):
    v &= (1 << 64) - 1
    if v >= (1 << 63):
        v -= 1 << 64
    return v


def _salt32(salt: int) -> int:
    s = int(salt)
    s = _wrap64_py((s >> 16 ^ s) * HASH_C)
    s = _wrap64_py((s >> 16 ^ s) * HASH_C)
    sv = s >> 16 ^ s
    sv &= (1 << 32) - 1
    if sv >= (1 << 31):
        sv -= 1 << 32
    return sv


SALTS = tuple(_salt32(n) for n in range(DIGESTS))


def _hash_mod(xv, salt):
    c = jnp.int32(HASH_C)
    k = xv ^ jnp.int32(salt)
    k = (k >> 16 ^ k) * c
    k = (k >> 16 ^ k) * c
    k = k >> 16 ^ k
    return k % jnp.int32(LUT_SIZE)


def _make_kernel(n_ids):
    per_w = n_ids // NW
    n_chunks = per_w // CH
    mesh = plsc.VectorSubcoreMesh(
        core_axis_name="c", subcore_axis_name="s",
        num_cores=NC, num_subcores=NS)

    def body(x_hbm, lut_hbm, out_hbm, x_v, idx_v, rows_v, sem):
        wid = lax.axis_index("s") * NC + lax.axis_index("c")
        base = wid * per_w

        def chunk(c, carry):
            off = base + c * CH
            pltpu.sync_copy(x_hbm.at[pl.ds(off, CH)], x_v)

            def hash_blk(j, carry2):
                for n in range(DIGESTS):
                    for l in range(GB // LANES):
                        xv = x_v[pl.ds(j * GB + l * LANES, LANES)]
                        idx_v[n * K + j, pl.ds(l * LANES, LANES)] = (
                            _hash_mod(xv, SALTS[n]))
                return carry2

            lax.fori_loop(0, K, hash_blk, 0)
            cps = []
            for n in range(DIGESTS):
                for j in range(K):
                    cps.append(pltpu.async_copy(
                        lut_hbm.at[idx_v.at[n * K + j]],
                        rows_v.at[n, pl.ds(j * GB, GB)], sem))
            for cp in cps:
                cp.wait()
            for n in range(DIGESTS):
                pltpu.sync_copy(rows_v.at[n],
                                out_hbm.at[pl.ds(off, CH), n])
            return carry

        lax.fori_loop(0, n_chunks, chunk, 0)

    return pl.kernel(
        body,
        out_type=jax.ShapeDtypeStruct((n_ids, DIGESTS, KEY_DIM),
                                      jnp.float32),
        mesh=mesh,
        compiler_params=pltpu.CompilerParams(use_tc_tiling_on_sc=False),
        scratch_types=[
            pltpu.VMEM((CH,), jnp.int32),
            pltpu.VMEM((DIGESTS * K, GB), jnp.int32),
            pltpu.VMEM((DIGESTS, CH, KEY_DIM), jnp.float32),
            pltpu.SemaphoreType.DMA,
        ],
    )


def kernel(x, lut):
    batch, hist = x.shape
    n_ids = batch * hist
    out = _make_kernel(n_ids)(x.reshape(n_ids), lut)
    return out.reshape(batch, hist, DIGESTS * KEY_DIM)


# speedup vs baseline: 2.1857x; 1.0619x over previous
import jax
import jax.numpy as jnp
from jax import lax
from jax.experimental import pallas as pl
from jax.experimental.pallas import tpu as pltpu
from jax.experimental.pallas import tpu_sc as plsc

LUT_SIZE = 1000000
KEY_DIM = 32
DIGESTS = 2
HASH_C = 73244475

NC = 2
NS = 16
NW = NC * NS
LANES = 16

CH = 1024
GB = 128
NG = DIGESTS * CH // GB


def _wrap64_py(v):
    v &= (1 << 64) - 1
    if v >= (1 << 63):
        v -= 1 << 64
    return v


def _salt32(salt: int) -> int:
    s = int(salt)
    s = _wrap64_py((s >> 16 ^ s) * HASH_C)
    s = _wrap64_py((s >> 16 ^ s) * HASH_C)
    sv = s >> 16 ^ s
    sv &= (1 << 32) - 1
    if sv >= (1 << 31):
        sv -= 1 << 32
    return sv


SALTS = tuple(_salt32(n) for n in range(DIGESTS))


def _hash_mod(xv, salt):
    c = jnp.int32(HASH_C)
    k = xv ^ jnp.int32(salt)
    k = (k >> 16 ^ k) * c
    k = (k >> 16 ^ k) * c
    k = k >> 16 ^ k
    return k % jnp.int32(LUT_SIZE)


def _make_kernel(n_ids):
    per_w = n_ids // NW
    n_chunks = per_w // CH
    mesh = plsc.VectorSubcoreMesh(
        core_axis_name="c", subcore_axis_name="s",
        num_cores=NC, num_subcores=NS)

    def body(x_hbm, lut_hbm, out_hbm, x_v, idx_v, rows_v, sem):
        wid = lax.axis_index("s") * NC + lax.axis_index("c")
        base = wid * per_w
        lane = lax.iota(jnp.int32, 16)

        def chunk(c, carry):
            off = base + c * CH
            pltpu.sync_copy(x_hbm.at[pl.ds(off, CH)], x_v)

            def hash_blk(i, carry2):
                xv = x_v[pl.ds(i * LANES, LANES)]
                p0 = (i * LANES + lane) * DIGESTS
                for n in range(DIGESTS):
                    p = p0 + n
                    plsc.store_scatter(
                        idx_v, [p >> 7, p & 127], _hash_mod(xv, SALTS[n]))
                return carry2

            lax.fori_loop(0, CH // LANES, hash_blk, 0)

            cps = [pltpu.async_copy(
                lut_hbm.at[idx_v.at[j]],
                rows_v.at[pl.ds(j * GB, GB)], sem) for j in range(NG)]
            for cp in cps:
                cp.wait()

            pltpu.sync_copy(rows_v,
                            out_hbm.at[pl.ds(DIGESTS * off, DIGESTS * CH)])
            return carry

        lax.fori_loop(0, n_chunks, chunk, 0)

    return pl.kernel(
        body,
        out_type=jax.ShapeDtypeStruct((DIGESTS * n_ids, KEY_DIM),
                                      jnp.float32),
        mesh=mesh,
        compiler_params=pltpu.CompilerParams(use_tc_tiling_on_sc=False,
                                             needs_layout_passes=False),
        scratch_types=[
            pltpu.VMEM((CH,), jnp.int32),
            pltpu.VMEM((NG, GB), jnp.int32),
            pltpu.VMEM((DIGESTS * CH, KEY_DIM), jnp.float32),
            pltpu.SemaphoreType.DMA,
        ],
    )


def kernel(x, lut):
    batch, hist = x.shape
    n_ids = batch * hist
    out = _make_kernel(n_ids)(x.reshape(n_ids), lut)
    return out.reshape(batch, hist, DIGESTS * KEY_DIM)
